# Initial kernel scaffold; baseline (speedup 1.0000x reference)
import jax
import jax.numpy as jnp
from jax.experimental import pallas as pl


def kernel(x, edge_index, gcn_norm, W_edge, b_edge, W_node, b_node):
    raise NotImplementedError("write your pallas kernel here")


# speedup vs baseline: 3.4540x; 3.4540x over previous
import dataclasses
import functools

import jax
import jax.numpy as jnp
from jax import lax
from jax.experimental import pallas as pl
from jax.experimental.pallas import tpu as pltpu
from jax.experimental.pallas import tpu_sc as plsc

N = 10000
E = 320000
D = 128
H = 128

NC = 2
NS = 16
CHUNK = 128
NCHUNKS = -(-E // (NC * NS * CHUNK))
EPW = NCHUNKS * CHUNK
E_PAD = NC * NS * EPW
E_HALF = E_PAD // NC
N_PAD = 10240
RPT = N_PAD // NS
LANES = 16


def _edge_mm_body(x_ref, w_ref, b_ref, o_ref):
    acc = jnp.dot(x_ref[...], w_ref[...], preferred_element_type=jnp.float32)
    o_ref[...] = jnp.maximum(acc + b_ref[...], 0.0)


def _node_mm_body(pp_ref, w_ref, b_ref, x_ref, o_ref):
    pooled = pp_ref[0, :N, :] + pp_ref[1, :N, :]
    acc = jnp.dot(pooled, w_ref[...], preferred_element_type=jnp.float32)
    o_ref[...] = jnp.maximum(acc + b_ref[...], 0.0) + x_ref[...]


def _sc_body(y_hbm, src_hbm, dst_hbm, nrm_hbm, zeros_hbm, out_hbm,
             src_v, dst_v, nrm_v, rows_v, pooled_sh, sem):
    c = lax.axis_index("c")
    s = lax.axis_index("s")

    pltpu.sync_copy(zeros_hbm.at[pl.ds(s * RPT, RPT)],
                    pooled_sh.at[pl.ds(s * RPT, RPT)])
    plsc.subcore_barrier()

    base0 = c * E_HALF + s * EPW

    @pl.loop(0, NCHUNKS)
    def _chunk(k):
        base = base0 + k * CHUNK
        pltpu.sync_copy(src_hbm.at[pl.ds(base, CHUNK)], src_v)
        pltpu.sync_copy(dst_hbm.at[pl.ds(base, CHUNK)], dst_v)
        pltpu.sync_copy(nrm_hbm.at[pl.ds(base, CHUNK)], nrm_v)
        pltpu.async_copy(y_hbm.at[src_v], rows_v, sem).wait()

        @pl.loop(0, CHUNK)
        def _edge(e):
            idx = jnp.full((LANES,), e, dtype=jnp.int32)
            nb = plsc.load_gather(nrm_v, [idx])
            for j in range(H // LANES):
                sl = (e, pl.ds(j * LANES, LANES))
                rows_v[sl] = rows_v[sl] * nb

        pltpu.sync_copy(rows_v, pooled_sh.at[dst_v], add=True)

    plsc.subcore_barrier()
    pltpu.sync_copy(pooled_sh.at[pl.ds(s * RPT, RPT)],
                    out_hbm.at[c, pl.ds(s * RPT, RPT)])


@jax.jit
def _run(x, src, dst, nrm, W_edge, b_edge, W_node, b_node):
    y = pl.pallas_call(
        _edge_mm_body,
        out_shape=jax.ShapeDtypeStruct((N, H), jnp.float32),
    )(x, W_edge, b_edge.reshape(1, H))

    zeros = jnp.zeros((N_PAD, H), dtype=jnp.float32)
    mesh = plsc.VectorSubcoreMesh(core_axis_name="c", subcore_axis_name="s")
    cp = pltpu.CompilerParams()
    if "needs_layout_passes" in pltpu.CompilerParams.__dataclass_fields__:
        cp = dataclasses.replace(cp, needs_layout_passes=False)
    partials = pl.kernel(
        _sc_body,
        out_type=jax.ShapeDtypeStruct((NC, N_PAD, H), jnp.float32),
        mesh=mesh,
        scratch_types=[
            pltpu.VMEM((CHUNK,), jnp.int32),
            pltpu.VMEM((CHUNK,), jnp.int32),
            pltpu.VMEM((CHUNK,), jnp.float32),
            pltpu.VMEM((CHUNK, H), jnp.float32),
            pltpu.VMEM_SHARED((N_PAD, H), jnp.float32),
            pltpu.SemaphoreType.DMA,
        ],
        compiler_params=cp,
    )(y, src, dst, nrm, zeros)

    out = pl.pallas_call(
        _node_mm_body,
        out_shape=jax.ShapeDtypeStruct((N, H), jnp.float32),
    )(partials, W_node, b_node.reshape(1, H), x)
    return out


def kernel(x, edge_index, gcn_norm, W_edge, b_edge, W_node, b_node):
    pad = E_PAD - E
    src = jnp.concatenate([edge_index[0], jnp.zeros((pad,), jnp.int32)])
    dst = jnp.concatenate([edge_index[1], jnp.zeros((pad,), jnp.int32)])
    nrm = jnp.concatenate([gcn_norm, jnp.zeros((pad,), jnp.float32)])
    return _run(x, src, dst, nrm, W_edge, b_edge, W_node, b_node)


# speedup vs baseline: 3.7118x; 1.0746x over previous
import dataclasses
import functools

import jax
import jax.numpy as jnp
from jax import lax
from jax.experimental import pallas as pl
from jax.experimental.pallas import tpu as pltpu
from jax.experimental.pallas import tpu_sc as plsc

N = 10000
E = 320000
D = 128
H = 128

NC = 2
NS = 16
CHUNK = 128
NCHUNKS = 80
EPW = NCHUNKS * CHUNK
E_PAD = NC * NS * EPW
N_PAD = 10240
RPT = N_PAD // NS
LANES = 16


def _edge_mm_body(x_ref, w_ref, b_ref, o_ref):
    acc = jnp.dot(x_ref[...], w_ref[...], preferred_element_type=jnp.float32)
    o_ref[...] = jnp.maximum(acc + b_ref[...], 0.0)


def _node_mm_body(pp_ref, w_ref, b_ref, x_ref, o_ref):
    pooled = pp_ref[0, :N, :] + pp_ref[1, :N, :]
    acc = jnp.dot(pooled, w_ref[...], preferred_element_type=jnp.float32)
    o_ref[...] = jnp.maximum(acc + b_ref[...], 0.0) + x_ref[...]


def _sc_body(y_hbm, meta_hbm, zeros_hbm, out_hbm,
             idx0, idx1, rows0, rows1, pooled_sh,
             sem_i0, sem_i1, sem_g0, sem_g1, sem_s0, sem_s1):
    c = lax.axis_index("c")
    s = lax.axis_index("s")
    w = c * NS + s
    row0 = w * NCHUNKS

    pltpu.sync_copy(zeros_hbm.at[pl.ds(s * RPT, RPT)],
                    pooled_sh.at[pl.ds(s * RPT, RPT)])
    plsc.subcore_barrier()

    def scale(rows, idx):
        @pl.loop(0, CHUNK, step=4)
        def _edge(e0):
            for u in range(4):
                e = e0 + u
                ie = jnp.full((LANES,), e, dtype=jnp.int32)
                i2 = jnp.full((LANES,), 2, dtype=jnp.int32)
                nb = plsc.bitcast(plsc.load_gather(idx, [i2, ie]), jnp.float32)
                for j in range(H // LANES):
                    sl = (e, pl.ds(j * LANES, LANES))
                    rows[sl] = rows[sl] * nb

    pltpu.sync_copy(meta_hbm.at[row0], idx0)
    pltpu.async_copy(y_hbm.at[idx0.at[0]], rows0, sem_g0)
    pltpu.sync_copy(meta_hbm.at[row0 + 1], idx1)
    pltpu.async_copy(y_hbm.at[idx1.at[0]], rows1, sem_g1)

    @pl.loop(0, NCHUNKS, step=2)
    def _pair(k):
        pltpu.make_async_copy(y_hbm.at[idx0.at[0]], rows0, sem_g0).wait()
        scale(rows0, idx0)
        pltpu.sync_copy(rows0, pooled_sh.at[idx0.at[1]], add=True)

        @pl.when(k + 2 < NCHUNKS)
        def _prefetch0():
            pltpu.sync_copy(meta_hbm.at[row0 + k + 2], idx0)
            pltpu.async_copy(y_hbm.at[idx0.at[0]], rows0, sem_g0)

        pltpu.make_async_copy(y_hbm.at[idx1.at[0]], rows1, sem_g1).wait()
        scale(rows1, idx1)
        pltpu.sync_copy(rows1, pooled_sh.at[idx1.at[1]], add=True)

        @pl.when(k + 3 < NCHUNKS)
        def _prefetch1():
            pltpu.sync_copy(meta_hbm.at[row0 + k + 3], idx1)
            pltpu.async_copy(y_hbm.at[idx1.at[0]], rows1, sem_g1)

    plsc.subcore_barrier()
    pltpu.sync_copy(pooled_sh.at[pl.ds(s * RPT, RPT)],
                    out_hbm.at[c, pl.ds(s * RPT, RPT)])


@jax.jit
def _run(x, meta, W_edge, b_edge, W_node, b_node):
    y = pl.pallas_call(
        _edge_mm_body,
        out_shape=jax.ShapeDtypeStruct((N, H), jnp.float32),
    )(x, W_edge, b_edge.reshape(1, H))

    zeros = jnp.zeros((N_PAD, H), dtype=jnp.float32)
    mesh = plsc.VectorSubcoreMesh(core_axis_name="c", subcore_axis_name="s")
    cp = pltpu.CompilerParams()
    if "needs_layout_passes" in pltpu.CompilerParams.__dataclass_fields__:
        cp = dataclasses.replace(cp, needs_layout_passes=False)
    partials = pl.kernel(
        _sc_body,
        out_type=jax.ShapeDtypeStruct((NC, N_PAD, H), jnp.float32),
        mesh=mesh,
        scratch_types=[
            pltpu.VMEM((3, CHUNK), jnp.int32),
            pltpu.VMEM((3, CHUNK), jnp.int32),
            pltpu.VMEM((CHUNK, H), jnp.float32),
            pltpu.VMEM((CHUNK, H), jnp.float32),
            pltpu.VMEM_SHARED((N_PAD, H), jnp.float32),
            pltpu.SemaphoreType.DMA,
            pltpu.SemaphoreType.DMA,
            pltpu.SemaphoreType.DMA,
            pltpu.SemaphoreType.DMA,
            pltpu.SemaphoreType.DMA,
            pltpu.SemaphoreType.DMA,
        ],
        compiler_params=cp,
    )(y, meta, zeros)

    out = pl.pallas_call(
        _node_mm_body,
        out_shape=jax.ShapeDtypeStruct((N, H), jnp.float32),
    )(partials, W_node, b_node.reshape(1, H), x)
    return out


def kernel(x, edge_index, gcn_norm, W_edge, b_edge, W_node, b_node):
    pad = E_PAD - E
    src = jnp.concatenate([edge_index[0], jnp.zeros((pad,), jnp.int32)])
    dst = jnp.concatenate([edge_index[1], jnp.zeros((pad,), jnp.int32)])
    nrm = jnp.concatenate([gcn_norm, jnp.zeros((pad,), jnp.float32)])
    nrm_i = lax.bitcast_convert_type(nrm, jnp.int32)
    nrows = E_PAD // CHUNK
    meta = jnp.stack([src.reshape(nrows, CHUNK),
                      dst.reshape(nrows, CHUNK),
                      nrm_i.reshape(nrows, CHUNK)], axis=1)
    return _run(x, meta, W_edge, b_edge, W_node, b_node)


# speedup vs baseline: 7.9223x; 2.1344x over previous
import dataclasses
import functools

import jax
import jax.numpy as jnp
from jax import lax
from jax.experimental import pallas as pl
from jax.experimental.pallas import tpu as pltpu
from jax.experimental.pallas import tpu_sc as plsc

N = 10000
E = 320000
D = 128
H = 128

NC = 2
NS = 16
CHUNK = 128
NCHUNKS = 80
EPW = NCHUNKS * CHUNK
E_PAD = NC * NS * EPW
N_PAD = 10240
RPT = N_PAD // NS
LANES = 16


def _edge_mm_body(x_ref, w_ref, b_ref, o_ref):
    acc = jnp.dot(x_ref[...], w_ref[...], preferred_element_type=jnp.float32)
    o_ref[...] = jnp.maximum(acc + b_ref[...], 0.0)


def _node_mm_body(pp_ref, w_ref, b_ref, x_ref, o_ref):
    pooled = pp_ref[0, :N, :] + pp_ref[1, :N, :]
    acc = jnp.dot(pooled, w_ref[...], preferred_element_type=jnp.float32)
    o_ref[...] = jnp.maximum(acc + b_ref[...], 0.0) + x_ref[...]


def _sc_body(y_hbm, meta_hbm, zeros_hbm, out_hbm,
             idx0, idx1, rows0, rows1, pooled_sh,
             sem_i0, sem_i1, sem_g0, sem_g1, sem_s0, sem_s1):
    c = lax.axis_index("c")
    s = lax.axis_index("s")
    w = c * NS + s
    row0 = w * NCHUNKS

    pltpu.sync_copy(zeros_hbm.at[pl.ds(s * RPT, RPT)],
                    pooled_sh.at[pl.ds(s * RPT, RPT)])
    plsc.subcore_barrier()

    def scale(rows, idx):
        @pl.loop(0, CHUNK, step=4)
        def _edge(e0):
            for u in range(4):
                e = e0 + u
                ie = jnp.full((LANES,), e, dtype=jnp.int32)
                i2 = jnp.full((LANES,), 2, dtype=jnp.int32)
                nb = plsc.bitcast(plsc.load_gather(idx, [i2, ie]), jnp.float32)
                for j in range(H // LANES):
                    sl = (e, pl.ds(j * LANES, LANES))
                    rows[sl] = rows[sl] * nb

    pltpu.sync_copy(meta_hbm.at[row0], idx0)
    pltpu.async_copy(y_hbm.at[idx0.at[0]], rows0, sem_g0)
    pltpu.sync_copy(meta_hbm.at[row0 + 1], idx1)
    pltpu.async_copy(y_hbm.at[idx1.at[0]], rows1, sem_g1)

    @pl.loop(0, NCHUNKS, step=2)
    def _pair(k):
        pltpu.make_async_copy(y_hbm.at[idx0.at[0]], rows0, sem_g0).wait()
        scale(rows0, idx0)
        pltpu.sync_copy(rows0, pooled_sh.at[idx0.at[1]], add=True)

        @pl.when(k + 2 < NCHUNKS)
        def _prefetch0():
            pltpu.sync_copy(meta_hbm.at[row0 + k + 2], idx0)
            pltpu.async_copy(y_hbm.at[idx0.at[0]], rows0, sem_g0)

        pltpu.make_async_copy(y_hbm.at[idx1.at[0]], rows1, sem_g1).wait()
        scale(rows1, idx1)
        pltpu.sync_copy(rows1, pooled_sh.at[idx1.at[1]], add=True)

        @pl.when(k + 3 < NCHUNKS)
        def _prefetch1():
            pltpu.sync_copy(meta_hbm.at[row0 + k + 3], idx1)
            pltpu.async_copy(y_hbm.at[idx1.at[0]], rows1, sem_g1)

    plsc.subcore_barrier()
    pltpu.sync_copy(pooled_sh.at[pl.ds(s * RPT, RPT)],
                    out_hbm.at[c, pl.ds(s * RPT, RPT)])


@jax.jit
def _run(x, meta, W_edge, b_edge, W_node, b_node):
    y = pl.pallas_call(
        _edge_mm_body,
        out_shape=jax.ShapeDtypeStruct((N, H), jnp.float32),
    )(x, W_edge, b_edge.reshape(1, H))

    zeros = jnp.zeros((N_PAD, H), dtype=jnp.float32)
    mesh = plsc.VectorSubcoreMesh(core_axis_name="c", subcore_axis_name="s")
    cp = pltpu.CompilerParams()
    if "needs_layout_passes" in pltpu.CompilerParams.__dataclass_fields__:
        cp = dataclasses.replace(cp, needs_layout_passes=False)
    partials = pl.kernel(
        _sc_body,
        out_type=jax.ShapeDtypeStruct((NC, N_PAD, H), jnp.float32),
        mesh=mesh,
        scratch_types=[
            pltpu.VMEM((3, CHUNK), jnp.int32),
            pltpu.VMEM((3, CHUNK), jnp.int32),
            pltpu.VMEM((CHUNK, H), jnp.float32),
            pltpu.VMEM((CHUNK, H), jnp.float32),
            pltpu.VMEM_SHARED((N_PAD, H), jnp.float32),
            pltpu.SemaphoreType.DMA,
            pltpu.SemaphoreType.DMA,
            pltpu.SemaphoreType.DMA,
            pltpu.SemaphoreType.DMA,
            pltpu.SemaphoreType.DMA,
            pltpu.SemaphoreType.DMA,
        ],
        compiler_params=cp,
    )(y, meta, zeros)

    out = pl.pallas_call(
        _node_mm_body,
        out_shape=jax.ShapeDtypeStruct((N, H), jnp.float32),
    )(partials, W_node, b_node.reshape(1, H), x)
    return out


def kernel(x, edge_index, gcn_norm, W_edge, b_edge, W_node, b_node):
    pad = E_PAD - E
    spread = (jnp.arange(pad, dtype=jnp.int32) * 41) % N
    src = jnp.concatenate([edge_index[0], spread])
    dst = jnp.concatenate([edge_index[1], spread])
    nrm = jnp.concatenate([gcn_norm, jnp.zeros((pad,), jnp.float32)])
    nrm_i = lax.bitcast_convert_type(nrm, jnp.int32)
    nrows = E_PAD // CHUNK
    meta = jnp.stack([src.reshape(nrows, CHUNK),
                      dst.reshape(nrows, CHUNK),
                      nrm_i.reshape(nrows, CHUNK)], axis=1)
    return _run(x, meta, W_edge, b_edge, W_node, b_node)


# speedup vs baseline: 9.1929x; 1.1604x over previous
import dataclasses
import functools

import jax
import jax.numpy as jnp
from jax import lax
from jax.experimental import pallas as pl
from jax.experimental.pallas import tpu as pltpu
from jax.experimental.pallas import tpu_sc as plsc

N = 10000
E = 320000
D = 128
H = 128

NC = 2
NS = 16
CHUNK = 128
NCHUNKS = 80
EPW = NCHUNKS * CHUNK
E_PAD = NC * NS * EPW
N_PAD = 10240
RPT = N_PAD // NS
LANES = 16


def _edge_mm_body(x_ref, w_ref, b_ref, o_ref):
    acc = jnp.dot(x_ref[...], w_ref[...], preferred_element_type=jnp.float32)
    o_ref[...] = jnp.maximum(acc + b_ref[...], 0.0)


def _node_mm_body(pp_ref, w_ref, b_ref, x_ref, o_ref):
    pooled = pp_ref[0, :N, :] + pp_ref[1, :N, :]
    acc = jnp.dot(pooled, w_ref[...], preferred_element_type=jnp.float32)
    o_ref[...] = jnp.maximum(acc + b_ref[...], 0.0) + x_ref[...]


def _sc_body(y_hbm, meta_hbm, zeros_hbm, out_hbm,
             idx0, idx1, idx2, idx3, rows0, rows1, pooled_sh,
             sem_i0, sem_i1, sem_i2, sem_i3, sem_g0, sem_g1, sem_s0, sem_s1):
    c = lax.axis_index("c")
    s = lax.axis_index("s")
    w = c * NS + s
    row0 = w * NCHUNKS

    pltpu.sync_copy(zeros_hbm.at[pl.ds(s * RPT, RPT)],
                    pooled_sh.at[pl.ds(s * RPT, RPT)])
    plsc.subcore_barrier()

    def scale(rows, idx):
        @pl.loop(0, CHUNK, step=4)
        def _edge(e0):
            for u in range(4):
                e = e0 + u
                ie = jnp.full((LANES,), e, dtype=jnp.int32)
                i2 = jnp.full((LANES,), 2, dtype=jnp.int32)
                nb = plsc.bitcast(plsc.load_gather(idx, [i2, ie]), jnp.float32)
                for j in range(H // LANES):
                    sl = (e, pl.ds(j * LANES, LANES))
                    rows[sl] = rows[sl] * nb

    idxs = (idx0, idx1, idx2, idx3)
    sem_is = (sem_i0, sem_i1, sem_i2, sem_i3)
    sem_gs = (sem_g0, sem_g1)
    sem_ss = (sem_s0, sem_s1)
    rowss = (rows0, rows1)

    pltpu.sync_copy(meta_hbm.at[row0], idx0)
    pltpu.sync_copy(meta_hbm.at[row0 + 1], idx1)
    pltpu.async_copy(y_hbm.at[idx0.at[0]], rows0, sem_g0)
    pltpu.async_copy(y_hbm.at[idx1.at[0]], rows1, sem_g1)
    pltpu.async_copy(meta_hbm.at[row0 + 2], idx2, sem_i2)
    pltpu.async_copy(meta_hbm.at[row0 + 3], idx3, sem_i3)

    def process(t, k):
        b, rows, idx = t % 2, rowss[t % 2], idxs[t % 4]
        pltpu.make_async_copy(y_hbm.at[idx.at[0]], rows, sem_gs[b]).wait()
        scale(rows, idx)
        pltpu.async_copy(rows, pooled_sh.at[idx.at[1]], sem_ss[b], add=True)

    def launch(t, k):
        b = t % 2
        rows, idx, idx_old = rowss[b], idxs[t % 4], idxs[(t - 2) % 4]

        @pl.when(k + t < NCHUNKS)
        def _():
            pltpu.make_async_copy(meta_hbm.at[row0 + k + t], idxs[t % 4],
                                  sem_is[t % 4]).wait()
            pltpu.make_async_copy(rows, pooled_sh.at[idx_old.at[1]],
                                  sem_ss[b]).wait()
            pltpu.async_copy(y_hbm.at[idx.at[0]], rows, sem_gs[b])

        @pl.when(k + t + 2 < NCHUNKS)
        def _():
            pltpu.async_copy(meta_hbm.at[row0 + k + t + 2], idx_old,
                             sem_is[(t - 2) % 4])

    @pl.loop(0, NCHUNKS, step=4)
    def _quad(k):
        process(0, k)
        process(1, k)
        launch(2, k)
        launch(3, k)
        process(2, k)
        process(3, k)
        launch(4, k)
        launch(5, k)

    pltpu.make_async_copy(rows0, pooled_sh.at[idx2.at[1]], sem_s0).wait()
    pltpu.make_async_copy(rows1, pooled_sh.at[idx3.at[1]], sem_s1).wait()
    plsc.subcore_barrier()
    pltpu.sync_copy(pooled_sh.at[pl.ds(s * RPT, RPT)],
                    out_hbm.at[c, pl.ds(s * RPT, RPT)])


@jax.jit
def _run(x, meta, W_edge, b_edge, W_node, b_node):
    y = pl.pallas_call(
        _edge_mm_body,
        out_shape=jax.ShapeDtypeStruct((N, H), jnp.float32),
    )(x, W_edge, b_edge.reshape(1, H))

    zeros = jnp.zeros((N_PAD, H), dtype=jnp.float32)
    mesh = plsc.VectorSubcoreMesh(core_axis_name="c", subcore_axis_name="s")
    cp = pltpu.CompilerParams()
    if "needs_layout_passes" in pltpu.CompilerParams.__dataclass_fields__:
        cp = dataclasses.replace(cp, needs_layout_passes=False)
    partials = pl.kernel(
        _sc_body,
        out_type=jax.ShapeDtypeStruct((NC, N_PAD, H), jnp.float32),
        mesh=mesh,
        scratch_types=(
            [pltpu.VMEM((3, CHUNK), jnp.int32)] * 4
            + [pltpu.VMEM((CHUNK, H), jnp.float32)] * 2
            + [pltpu.VMEM_SHARED((N_PAD, H), jnp.float32)]
            + [pltpu.SemaphoreType.DMA] * 8
        ),
        compiler_params=cp,
    )(y, meta, zeros)

    out = pl.pallas_call(
        _node_mm_body,
        out_shape=jax.ShapeDtypeStruct((N, H), jnp.float32),
    )(partials, W_node, b_node.reshape(1, H), x)
    return out


def kernel(x, edge_index, gcn_norm, W_edge, b_edge, W_node, b_node):
    pad = E_PAD - E
    spread = (jnp.arange(pad, dtype=jnp.int32) * 41) % N
    src = jnp.concatenate([edge_index[0], spread])
    dst = jnp.concatenate([edge_index[1], spread])
    nrm = jnp.concatenate([gcn_norm, jnp.zeros((pad,), jnp.float32)])
    nrm_i = lax.bitcast_convert_type(nrm, jnp.int32)
    nrows = E_PAD // CHUNK
    meta = jnp.stack([src.reshape(nrows, CHUNK),
                      dst.reshape(nrows, CHUNK),
                      nrm_i.reshape(nrows, CHUNK)], axis=1)
    return _run(x, meta, W_edge, b_edge, W_node, b_node)


# speedup vs baseline: 10.0868x; 1.0972x over previous
import dataclasses
import functools

import jax
import jax.numpy as jnp
from jax import lax
from jax.experimental import pallas as pl
from jax.experimental.pallas import tpu as pltpu
from jax.experimental.pallas import tpu_sc as plsc

N = 10000
E = 320000
D = 128
H = 128

NC = 2
NS = 16
CHUNK = 128
NCHUNKS = 80
EPW = NCHUNKS * CHUNK
E_PAD = NC * NS * EPW
N_PAD = 10240
RPT = N_PAD // NS
LANES = 16


def _edge_mm_body(x_ref, w_ref, b_ref, o_ref):
    acc = jnp.dot(x_ref[...], w_ref[...], preferred_element_type=jnp.float32)
    o_ref[...] = jnp.maximum(acc + b_ref[...], 0.0)


def _node_mm_body(pp_ref, w_ref, b_ref, x_ref, o_ref):
    pooled = pp_ref[0, :N, :] + pp_ref[1, :N, :]
    acc = jnp.dot(pooled, w_ref[...], preferred_element_type=jnp.float32)
    o_ref[...] = jnp.maximum(acc + b_ref[...], 0.0) + x_ref[...]


def _sc_body(y_hbm, meta_hbm, zeros_hbm, out_hbm,
             idx0, idx1, idx2, idx3, rows0, rows1, pooled_sh,
             sem_i0, sem_i1, sem_i2, sem_i3, sem_g0, sem_g1, sem_s0, sem_s1):
    c = lax.axis_index("c")
    s = lax.axis_index("s")
    w = c * NS + s
    row0 = w * NCHUNKS

    pltpu.sync_copy(zeros_hbm.at[pl.ds(s * RPT, RPT)],
                    pooled_sh.at[pl.ds(s * RPT, RPT)])
    plsc.subcore_barrier()

    i2 = jnp.full((LANES,), 2, dtype=jnp.int32)

    def scale(rows, idx):
        @plsc.parallel_loop(0, CHUNK, step=1, unroll=4)
        def _edge(e):
            ie = jnp.full((LANES,), e, dtype=jnp.int32)
            nb = plsc.bitcast(plsc.load_gather(idx, [i2, ie]), jnp.float32)
            for j in range(H // LANES):
                sl = (e, pl.ds(j * LANES, LANES))
                rows[sl] = rows[sl] * nb

    idxs = (idx0, idx1, idx2, idx3)
    sem_is = (sem_i0, sem_i1, sem_i2, sem_i3)
    sem_gs = (sem_g0, sem_g1)
    sem_ss = (sem_s0, sem_s1)
    rowss = (rows0, rows1)

    pltpu.sync_copy(meta_hbm.at[row0], idx0)
    pltpu.sync_copy(meta_hbm.at[row0 + 1], idx1)
    pltpu.async_copy(y_hbm.at[idx0.at[0]], rows0, sem_g0)
    pltpu.async_copy(y_hbm.at[idx1.at[0]], rows1, sem_g1)
    pltpu.async_copy(meta_hbm.at[row0 + 2], idx2, sem_i2)
    pltpu.async_copy(meta_hbm.at[row0 + 3], idx3, sem_i3)

    def process(t, k):
        b, rows, idx = t % 2, rowss[t % 2], idxs[t % 4]
        pltpu.make_async_copy(y_hbm.at[idx.at[0]], rows, sem_gs[b]).wait()
        scale(rows, idx)
        pltpu.async_copy(rows, pooled_sh.at[idx.at[1]], sem_ss[b], add=True)

    def launch(t, k):
        b = t % 2
        rows, idx, idx_old = rowss[b], idxs[t % 4], idxs[(t - 2) % 4]

        @pl.when(k + t < NCHUNKS)
        def _():
            pltpu.make_async_copy(meta_hbm.at[row0 + k + t], idxs[t % 4],
                                  sem_is[t % 4]).wait()
            pltpu.make_async_copy(rows, pooled_sh.at[idx_old.at[1]],
                                  sem_ss[b]).wait()
            pltpu.async_copy(y_hbm.at[idx.at[0]], rows, sem_gs[b])

        @pl.when(k + t + 2 < NCHUNKS)
        def _():
            pltpu.async_copy(meta_hbm.at[row0 + k + t + 2], idx_old,
                             sem_is[(t - 2) % 4])

    @pl.loop(0, NCHUNKS, step=4)
    def _quad(k):
        process(0, k)
        process(1, k)
        launch(2, k)
        launch(3, k)
        process(2, k)
        process(3, k)
        launch(4, k)
        launch(5, k)

    pltpu.make_async_copy(rows0, pooled_sh.at[idx2.at[1]], sem_s0).wait()
    pltpu.make_async_copy(rows1, pooled_sh.at[idx3.at[1]], sem_s1).wait()
    plsc.subcore_barrier()
    pltpu.sync_copy(pooled_sh.at[pl.ds(s * RPT, RPT)],
                    out_hbm.at[c, pl.ds(s * RPT, RPT)])


@jax.jit
def _run(x, meta, W_edge, b_edge, W_node, b_node):
    y = pl.pallas_call(
        _edge_mm_body,
        out_shape=jax.ShapeDtypeStruct((N, H), jnp.float32),
    )(x, W_edge, b_edge.reshape(1, H))

    zeros = jnp.zeros((N_PAD, H), dtype=jnp.float32)
    mesh = plsc.VectorSubcoreMesh(core_axis_name="c", subcore_axis_name="s")
    cp = pltpu.CompilerParams()
    if "needs_layout_passes" in pltpu.CompilerParams.__dataclass_fields__:
        cp = dataclasses.replace(cp, needs_layout_passes=False)
    partials = pl.kernel(
        _sc_body,
        out_type=jax.ShapeDtypeStruct((NC, N_PAD, H), jnp.float32),
        mesh=mesh,
        scratch_types=(
            [pltpu.VMEM((3, CHUNK), jnp.int32)] * 4
            + [pltpu.VMEM((CHUNK, H), jnp.float32)] * 2
            + [pltpu.VMEM_SHARED((N_PAD, H), jnp.float32)]
            + [pltpu.SemaphoreType.DMA] * 8
        ),
        compiler_params=cp,
    )(y, meta, zeros)

    out = pl.pallas_call(
        _node_mm_body,
        out_shape=jax.ShapeDtypeStruct((N, H), jnp.float32),
    )(partials, W_node, b_node.reshape(1, H), x)
    return out


def kernel(x, edge_index, gcn_norm, W_edge, b_edge, W_node, b_node):
    pad = E_PAD - E
    spread = (jnp.arange(pad, dtype=jnp.int32) * 41) % N
    src = jnp.concatenate([edge_index[0], spread])
    dst = jnp.concatenate([edge_index[1], spread])
    nrm = jnp.concatenate([gcn_norm, jnp.zeros((pad,), jnp.float32)])
    nrm_i = lax.bitcast_convert_type(nrm, jnp.int32)
    nrows = E_PAD // CHUNK
    meta = jnp.stack([src.reshape(nrows, CHUNK),
                      dst.reshape(nrows, CHUNK),
                      nrm_i.reshape(nrows, CHUNK)], axis=1)
    return _run(x, meta, W_edge, b_edge, W_node, b_node)


# speedup vs baseline: 10.4369x; 1.0347x over previous
import dataclasses
import functools

import jax
import jax.numpy as jnp
from jax import lax
from jax.experimental import pallas as pl
from jax.experimental.pallas import tpu as pltpu
from jax.experimental.pallas import tpu_sc as plsc

N = 10000
E = 320000
D = 128
H = 128

NC = 2
NS = 16
CHUNK = 128
NCHUNKS = 80
EPW = NCHUNKS * CHUNK
E_PAD = NC * NS * EPW
N_PAD = 10240
RPT = N_PAD // NS
LANES = 16


def _edge_mm_body(x_ref, w_ref, b_ref, src_ref, dst_ref, nrm_ref,
                  y_ref, meta_ref):
    acc = jnp.dot(x_ref[...], w_ref[...], preferred_element_type=jnp.float32)
    y_ref[...] = jnp.maximum(acc + b_ref[...], 0.0)
    nreal = E // CHUNK
    meta_ref[:nreal, 0, :] = src_ref[...]
    meta_ref[:nreal, 1, :] = dst_ref[...]
    meta_ref[:nreal, 2, :] = lax.bitcast_convert_type(nrm_ref[...], jnp.int32)
    npad = E_PAD // CHUNK - nreal
    r = lax.broadcasted_iota(jnp.int32, (npad, CHUNK), 0)
    cidx = lax.broadcasted_iota(jnp.int32, (npad, CHUNK), 1)
    spread = ((r * CHUNK + cidx) * 41) % N
    meta_ref[nreal:, 0, :] = spread
    meta_ref[nreal:, 1, :] = spread
    meta_ref[nreal:, 2, :] = jnp.zeros((npad, CHUNK), jnp.int32)


def _node_mm_body(pp_ref, w_ref, b_ref, x_ref, o_ref):
    pooled = pp_ref[0, :N, :] + pp_ref[1, :N, :]
    acc = jnp.dot(pooled, w_ref[...], preferred_element_type=jnp.float32)
    o_ref[...] = jnp.maximum(acc + b_ref[...], 0.0) + x_ref[...]


def _sc_body(y_hbm, meta_hbm, out_hbm,
             idx0, idx1, idx2, idx3, rows0, rows1, pooled_sh,
             sem_i0, sem_i1, sem_i2, sem_i3, sem_g0, sem_g1, sem_s0, sem_s1):
    c = lax.axis_index("c")
    s = lax.axis_index("s")
    w = c * NS + s
    row0 = w * NCHUNKS

    @plsc.parallel_loop(0, CHUNK)
    def _zero(i):
        for j in range(H // LANES):
            rows0[i, pl.ds(j * LANES, LANES)] = jnp.zeros((LANES,), jnp.float32)

    for r in range(RPT // CHUNK):
        pltpu.sync_copy(rows0, pooled_sh.at[pl.ds(s * RPT + r * CHUNK, CHUNK)])
    plsc.subcore_barrier()

    i2 = jnp.full((LANES,), 2, dtype=jnp.int32)

    def scale(rows, idx):
        @plsc.parallel_loop(0, CHUNK, step=1, unroll=4)
        def _edge(e):
            ie = jnp.full((LANES,), e, dtype=jnp.int32)
            nb = plsc.bitcast(plsc.load_gather(idx, [i2, ie]), jnp.float32)
            for j in range(H // LANES):
                sl = (e, pl.ds(j * LANES, LANES))
                rows[sl] = rows[sl] * nb

    idxs = (idx0, idx1, idx2, idx3)
    sem_is = (sem_i0, sem_i1, sem_i2, sem_i3)
    sem_gs = (sem_g0, sem_g1)
    sem_ss = (sem_s0, sem_s1)
    rowss = (rows0, rows1)

    pltpu.sync_copy(meta_hbm.at[row0], idx0)
    pltpu.sync_copy(meta_hbm.at[row0 + 1], idx1)
    pltpu.async_copy(y_hbm.at[idx0.at[0]], rows0, sem_g0)
    pltpu.async_copy(y_hbm.at[idx1.at[0]], rows1, sem_g1)
    pltpu.async_copy(meta_hbm.at[row0 + 2], idx2, sem_i2)
    pltpu.async_copy(meta_hbm.at[row0 + 3], idx3, sem_i3)

    def process(t, k):
        b, rows, idx = t % 2, rowss[t % 2], idxs[t % 4]
        pltpu.make_async_copy(y_hbm.at[idx.at[0]], rows, sem_gs[b]).wait()
        scale(rows, idx)
        pltpu.async_copy(rows, pooled_sh.at[idx.at[1]], sem_ss[b], add=True)

    def launch(t, k):
        b = t % 2
        rows, idx, idx_old = rowss[b], idxs[t % 4], idxs[(t - 2) % 4]

        @pl.when(k + t < NCHUNKS)
        def _():
            pltpu.make_async_copy(meta_hbm.at[row0 + k + t], idxs[t % 4],
                                  sem_is[t % 4]).wait()
            pltpu.make_async_copy(rows, pooled_sh.at[idx_old.at[1]],
                                  sem_ss[b]).wait()
            pltpu.async_copy(y_hbm.at[idx.at[0]], rows, sem_gs[b])

        @pl.when(k + t + 2 < NCHUNKS)
        def _():
            pltpu.async_copy(meta_hbm.at[row0 + k + t + 2], idx_old,
                             sem_is[(t - 2) % 4])

    @pl.loop(0, NCHUNKS, step=4)
    def _quad(k):
        process(0, k)
        process(1, k)
        launch(2, k)
        launch(3, k)
        process(2, k)
        process(3, k)
        launch(4, k)
        launch(5, k)

    pltpu.make_async_copy(rows0, pooled_sh.at[idx2.at[1]], sem_s0).wait()
    pltpu.make_async_copy(rows1, pooled_sh.at[idx3.at[1]], sem_s1).wait()
    plsc.subcore_barrier()
    pltpu.sync_copy(pooled_sh.at[pl.ds(s * RPT, RPT)],
                    out_hbm.at[c, pl.ds(s * RPT, RPT)])


@jax.jit
def _run(x, src2d, dst2d, nrm2d, W_edge, b_edge, W_node, b_node):
    y, meta = pl.pallas_call(
        _edge_mm_body,
        out_shape=(
            jax.ShapeDtypeStruct((N, H), jnp.float32),
            jax.ShapeDtypeStruct((E_PAD // CHUNK, 3, CHUNK), jnp.int32),
        ),
    )(x, W_edge, b_edge.reshape(1, H), src2d, dst2d, nrm2d)

    mesh = plsc.VectorSubcoreMesh(core_axis_name="c", subcore_axis_name="s")
    cp = pltpu.CompilerParams()
    if "needs_layout_passes" in pltpu.CompilerParams.__dataclass_fields__:
        cp = dataclasses.replace(cp, needs_layout_passes=False)
    partials = pl.kernel(
        _sc_body,
        out_type=jax.ShapeDtypeStruct((NC, N_PAD, H), jnp.float32),
        mesh=mesh,
        scratch_types=(
            [pltpu.VMEM((3, CHUNK), jnp.int32)] * 4
            + [pltpu.VMEM((CHUNK, H), jnp.float32)] * 2
            + [pltpu.VMEM_SHARED((N_PAD, H), jnp.float32)]
            + [pltpu.SemaphoreType.DMA] * 8
        ),
        compiler_params=cp,
    )(y, meta)

    out = pl.pallas_call(
        _node_mm_body,
        out_shape=jax.ShapeDtypeStruct((N, H), jnp.float32),
    )(partials, W_node, b_node.reshape(1, H), x)
    return out


def kernel(x, edge_index, gcn_norm, W_edge, b_edge, W_node, b_node):
    nreal = E // CHUNK
    src2d = edge_index[0].reshape(nreal, CHUNK)
    dst2d = edge_index[1].reshape(nreal, CHUNK)
    nrm2d = gcn_norm.reshape(nreal, CHUNK)
    return _run(x, src2d, dst2d, nrm2d, W_edge, b_edge, W_node, b_node)


# speedup vs baseline: 11.9016x; 1.1403x over previous
import dataclasses
import functools

import jax
import jax.numpy as jnp
from jax import lax
from jax.experimental import pallas as pl
from jax.experimental.pallas import tpu as pltpu
from jax.experimental.pallas import tpu_sc as plsc

N = 10000
E = 320000
D = 128
H = 128

NC = 2
NS = 16
CHUNK = 120
NCHUNKS = 84
EPW = NCHUNKS * CHUNK
E_PAD = NC * NS * EPW
N_PAD = 10240
RPT = N_PAD // NS
LANES = 16
NR = 3
NI = 6


def _edge_mm_body(x_ref, w_ref, b_ref, y_ref):
    acc = jnp.dot(x_ref[...], w_ref[...], preferred_element_type=jnp.float32)
    y_ref[...] = jnp.maximum(acc + b_ref[...], 0.0)


def _node_mm_body(pp_ref, w_ref, b_ref, x_ref, o_ref):
    pooled = pp_ref[0, :N, :] + pp_ref[1, :N, :]
    acc = jnp.dot(pooled, w_ref[...], preferred_element_type=jnp.float32)
    o_ref[...] = jnp.maximum(acc + b_ref[...], 0.0) + x_ref[...]


def _sc_body(y_hbm, meta_hbm, out_hbm, *scratch):
    idxs = scratch[0:NI]
    rowss = scratch[NI:NI + NR]
    pooled_sh = scratch[NI + NR]
    sems = scratch[NI + NR + 1:]
    sem_is = sems[0:NI]
    sem_gs = sems[NI:NI + NR]
    sem_ss = sems[NI + NR:NI + 2 * NR]

    c = lax.axis_index("c")
    s = lax.axis_index("s")
    w = c * NS + s
    row0 = w * NCHUNKS

    rows0 = rowss[0]

    @plsc.parallel_loop(0, CHUNK)
    def _zero(i):
        for j in range(H // LANES):
            rows0[i, pl.ds(j * LANES, LANES)] = jnp.zeros((LANES,), jnp.float32)

    for r in range(RPT // CHUNK):
        pltpu.sync_copy(rows0, pooled_sh.at[pl.ds(s * RPT + r * CHUNK, CHUNK)])
    _REM = RPT % CHUNK
    if _REM:
        pltpu.sync_copy(
            rows0.at[pl.ds(0, _REM)],
            pooled_sh.at[pl.ds(s * RPT + (RPT // CHUNK) * CHUNK, _REM)])
    plsc.subcore_barrier()

    i2 = jnp.full((LANES,), 2, dtype=jnp.int32)

    def scale(rows, idx):
        @plsc.parallel_loop(0, CHUNK, step=1, unroll=4)
        def _edge(e):
            ie = jnp.full((LANES,), e, dtype=jnp.int32)
            nb = plsc.bitcast(plsc.load_gather(idx, [i2, ie]), jnp.float32)
            for j in range(H // LANES):
                sl = (e, pl.ds(j * LANES, LANES))
                rows[sl] = rows[sl] * nb

    pltpu.sync_copy(meta_hbm.at[row0], idxs[0])
    pltpu.sync_copy(meta_hbm.at[row0 + 1], idxs[1])
    pltpu.async_copy(y_hbm.at[idxs[0].at[0]], rowss[0], sem_gs[0])
    pltpu.async_copy(y_hbm.at[idxs[1].at[0]], rowss[1], sem_gs[1])
    for q in (2, 3, 4):
        pltpu.async_copy(meta_hbm.at[row0 + q], idxs[q], sem_is[q])

    def process(u, k):
        b, t = u % NR, u % NI
        rows, idx = rowss[b], idxs[t]
        pltpu.make_async_copy(y_hbm.at[idx.at[0]], rows, sem_gs[b]).wait()
        scale(rows, idx)
        pltpu.async_copy(rows, pooled_sh.at[idx.at[1]], sem_ss[b], add=True)

    def launch(u, k):
        q = k + u
        b, t, t3 = u % NR, u % NI, (u + 3) % NI
        rows, idx, idx_old = rowss[b], idxs[t], idxs[t3]

        @pl.when(jnp.logical_and(q >= 3, q < NCHUNKS))
        def _():
            pltpu.make_async_copy(rows, pooled_sh.at[idx_old.at[1]],
                                  sem_ss[b]).wait()

        @pl.when(q < NCHUNKS)
        def _():
            pltpu.make_async_copy(meta_hbm.at[row0 + q], idx, sem_is[t]).wait()
            pltpu.async_copy(y_hbm.at[idx.at[0]], rows, sem_gs[b])

        @pl.when(q + 3 < NCHUNKS)
        def _():
            pltpu.async_copy(meta_hbm.at[row0 + q + 3], idx_old, sem_is[t3])

    @pl.loop(0, NCHUNKS, step=NI)
    def _six(k):
        for u in range(NI):
            process(u, k)
            launch(u + 2, k)

    for j in range(NR):
        m = NCHUNKS - NR + j
        pltpu.make_async_copy(rowss[m % NR], pooled_sh.at[idxs[m % NI].at[1]],
                              sem_ss[m % NR]).wait()
    plsc.subcore_barrier()
    pltpu.sync_copy(pooled_sh.at[pl.ds(s * RPT, RPT)],
                    out_hbm.at[c, pl.ds(s * RPT, RPT)])


@jax.jit
def _run(x, meta, W_edge, b_edge, W_node, b_node):
    y = pl.pallas_call(
        _edge_mm_body,
        out_shape=jax.ShapeDtypeStruct((N, H), jnp.float32),
    )(x, W_edge, b_edge.reshape(1, H))

    mesh = plsc.VectorSubcoreMesh(core_axis_name="c", subcore_axis_name="s")
    cp = pltpu.CompilerParams()
    if "needs_layout_passes" in pltpu.CompilerParams.__dataclass_fields__:
        cp = dataclasses.replace(cp, needs_layout_passes=False)
    partials = pl.kernel(
        _sc_body,
        out_type=jax.ShapeDtypeStruct((NC, N_PAD, H), jnp.float32),
        mesh=mesh,
        scratch_types=(
            [pltpu.VMEM((3, CHUNK), jnp.int32)] * NI
            + [pltpu.VMEM((CHUNK, H), jnp.float32)] * NR
            + [pltpu.VMEM_SHARED((N_PAD, H), jnp.float32)]
            + [pltpu.SemaphoreType.DMA] * (NI + 2 * NR)
        ),
        compiler_params=cp,
    )(y, meta)

    out = pl.pallas_call(
        _node_mm_body,
        out_shape=jax.ShapeDtypeStruct((N, H), jnp.float32),
    )(partials, W_node, b_node.reshape(1, H), x)
    return out


def kernel(x, edge_index, gcn_norm, W_edge, b_edge, W_node, b_node):
    pad = E_PAD - E
    spread = (jnp.arange(pad, dtype=jnp.int32) * 41) % N
    src = jnp.concatenate([edge_index[0], spread])
    dst = jnp.concatenate([edge_index[1], spread])
    nrm = jnp.concatenate([gcn_norm, jnp.zeros((pad,), jnp.float32)])
    nrm_i = lax.bitcast_convert_type(nrm, jnp.int32)
    nrows = E_PAD // CHUNK
    meta = jnp.stack([src.reshape(nrows, CHUNK),
                      dst.reshape(nrows, CHUNK),
                      nrm_i.reshape(nrows, CHUNK)], axis=1)
    return _run(x, meta, W_edge, b_edge, W_node, b_node)
